# Initial kernel scaffold; baseline (speedup 1.0000x reference)
import jax
import jax.numpy as jnp
from jax.experimental import pallas as pl


def kernel(from_node, to_node, emb, fc1_w, fc1_b, fc2_w, fc2_b):
    raise NotImplementedError("write your pallas kernel here")


# speedup vs baseline: 1.3587x; 1.3587x over previous
import functools

import jax
import jax.numpy as jnp
from jax import lax
from jax.experimental import pallas as pl
from jax.experimental.pallas import tpu as pltpu
from jax.experimental.pallas import tpu_sc as plsc

EMBED_DIM = 32
HIDDEN_DIM = 64
BATCH = 16384

NUM_CORES = 2
NUM_SUBCORES = 16
NUM_WORKERS = NUM_CORES * NUM_SUBCORES
B_PER_W = BATCH // NUM_WORKERS
CHUNK = 128
N_CHUNKS = B_PER_W // CHUNK

_sc_mesh = plsc.VectorSubcoreMesh(core_axis_name="c", subcore_axis_name="s")


@functools.partial(
    pl.kernel,
    mesh=_sc_mesh,
    out_type=(
        jax.ShapeDtypeStruct((BATCH, EMBED_DIM), jnp.float32),
        jax.ShapeDtypeStruct((BATCH, EMBED_DIM), jnp.float32),
    ),
    scratch_types=[
        pltpu.VMEM((B_PER_W,), jnp.int32),
        pltpu.VMEM((B_PER_W,), jnp.int32),
        pltpu.VMEM((B_PER_W, EMBED_DIM), jnp.float32),
        pltpu.VMEM((B_PER_W, EMBED_DIM), jnp.float32),
        pltpu.SemaphoreType.DMA,
    ],
    compiler_params=pltpu.CompilerParams(use_tc_tiling_on_sc=False),
)
def _gather_pairs(from_hbm, to_hbm, emb_hbm, out_f, out_t,
                  idx_f, idx_t, rows_f, rows_t, sem):
    wid = lax.axis_index("s") * NUM_CORES + lax.axis_index("c")
    base = wid * B_PER_W
    pltpu.sync_copy(from_hbm.at[pl.ds(base, B_PER_W)], idx_f)
    pltpu.sync_copy(to_hbm.at[pl.ds(base, B_PER_W)], idx_t)
    copies = []
    for j in range(N_CHUNKS):
        s = pl.ds(j * CHUNK, CHUNK)
        copies.append(pltpu.async_copy(emb_hbm.at[idx_f.at[s]], rows_f.at[s], sem))
        copies.append(pltpu.async_copy(emb_hbm.at[idx_t.at[s]], rows_t.at[s], sem))
    for c in copies:
        c.wait()
    pltpu.sync_copy(rows_f, out_f.at[pl.ds(base, B_PER_W)])
    pltpu.sync_copy(rows_t, out_t.at[pl.ds(base, B_PER_W)])


def _mlp_body(f_ref, t_ref, w1a_ref, w1b_ref, b1_ref, w2_ref, b2_ref, out_ref):
    h = jnp.dot(f_ref[...], w1a_ref[...], preferred_element_type=jnp.float32)
    h = h + jnp.dot(t_ref[...], w1b_ref[...], preferred_element_type=jnp.float32)
    h = jnp.maximum(h + b1_ref[...], 0.0)
    logit = jnp.sum(h * w2_ref[...], axis=1, keepdims=True) + b2_ref[...]
    out_ref[...] = jax.nn.sigmoid(logit)


def _mlp(f_rows, t_rows, w1a, w1b, b1, w2, b2, block_m=2048):
    grid = (BATCH // block_m,)
    return pl.pallas_call(
        _mlp_body,
        grid=grid,
        in_specs=[
            pl.BlockSpec((block_m, EMBED_DIM), lambda i: (i, 0)),
            pl.BlockSpec((block_m, EMBED_DIM), lambda i: (i, 0)),
            pl.BlockSpec((EMBED_DIM, HIDDEN_DIM), lambda i: (0, 0)),
            pl.BlockSpec((EMBED_DIM, HIDDEN_DIM), lambda i: (0, 0)),
            pl.BlockSpec((1, HIDDEN_DIM), lambda i: (0, 0)),
            pl.BlockSpec((1, HIDDEN_DIM), lambda i: (0, 0)),
            pl.BlockSpec((1, 1), lambda i: (0, 0)),
        ],
        out_specs=pl.BlockSpec((block_m, 1), lambda i: (i, 0)),
        out_shape=jax.ShapeDtypeStruct((BATCH, 1), jnp.float32),
    )(f_rows, t_rows, w1a, w1b, b1, w2, b2)


def kernel(from_node, to_node, emb, fc1_w, fc1_b, fc2_w, fc2_b):
    f_rows, t_rows = _gather_pairs(
        from_node.astype(jnp.int32), to_node.astype(jnp.int32), emb)
    w1a = fc1_w[:, :EMBED_DIM].T
    w1b = fc1_w[:, EMBED_DIM:].T
    b1 = fc1_b.reshape(1, HIDDEN_DIM)
    w2 = fc2_w.reshape(1, HIDDEN_DIM)
    b2 = fc2_b.reshape(1, 1)
    return _mlp(f_rows, t_rows, w1a, w1b, b1, w2, b2)
